# baseline (speedup 1.0000x reference)
import functools

import jax
import jax.numpy as jnp
from jax import lax
from jax.experimental import pallas as pl
from jax.experimental.pallas import tpu as pltpu
from jax.experimental.pallas import tpu_sc as plsc

_NC = 2
_NS = 16
_NW = _NC * _NS


def _silu(x):
    return x * jax.nn.sigmoid(x)


def _tc_precompute(h, Wa, Wb, tile_n):
    N, D = h.shape

    def body(h_ref, wa_ref, wb_ref, a_ref, b_ref):
        hx = h_ref[...]
        a_ref[...] = jnp.dot(hx, wa_ref[...], preferred_element_type=jnp.float32,
                             precision=jax.lax.Precision.HIGHEST)
        b_ref[...] = jnp.dot(hx, wb_ref[...], preferred_element_type=jnp.float32,
                             precision=jax.lax.Precision.HIGHEST)

    return pl.pallas_call(
        body,
        grid=(N // tile_n,),
        in_specs=[
            pl.BlockSpec((tile_n, D), lambda i: (i, 0)),
            pl.BlockSpec((D, D), lambda i: (0, 0)),
            pl.BlockSpec((D, D), lambda i: (0, 0)),
        ],
        out_specs=[
            pl.BlockSpec((tile_n, D), lambda i: (i, 0)),
            pl.BlockSpec((tile_n, D), lambda i: (i, 0)),
        ],
        out_shape=[jax.ShapeDtypeStruct((N, D), jnp.float32)] * 2,
    )(h, Wa, Wb)


def _sc_gather_add(row, col, A, B, chunk):
    E = row.shape[0]
    N, D = A.shape
    epw = E // _NW
    n_ch = epw // chunk
    mesh = plsc.VectorSubcoreMesh(core_axis_name="c", subcore_axis_name="s")

    @functools.partial(
        pl.kernel,
        out_type=jax.ShapeDtypeStruct((E, D), jnp.float32),
        mesh=mesh,
        scratch_types=(
            [pltpu.VMEM((chunk,), jnp.int32)] * 8
            + [pltpu.VMEM((chunk, D), jnp.float32)] * 8
            + [pltpu.SemaphoreType.DMA] * 12
        ),
    )
    def k(row_h, col_h, a_h, b_h, s_h,
          ir0, ir1, ir2, ir3, ic0, ic1, ic2, ic3,
          ba0, ba1, ba2, ba3, bb0, bb1, bb2, bb3,
          sa0, sa1, sa2, sa3, sb0, sb1, sb2, sb3, ss0, ss1, ss2, ss3):
        cid = lax.axis_index("c")
        sid = lax.axis_index("s")
        base = (sid * _NC + cid) * epw
        irs, ics = [ir0, ir1, ir2, ir3], [ic0, ic1, ic2, ic3]
        bas, bbs = [ba0, ba1, ba2, ba3], [bb0, bb1, bb2, bb3]
        sas, sbs = [sa0, sa1, sa2, sa3], [sb0, sb1, sb2, sb3]
        sss = [ss0, ss1, ss2, ss3]

        def issue(i, s):
            off = base + i * chunk
            pltpu.sync_copy(row_h.at[pl.ds(off, chunk)], irs[s])
            pltpu.sync_copy(col_h.at[pl.ds(off, chunk)], ics[s])
            pltpu.async_copy(a_h.at[irs[s]], bas[s], sas[s])
            pltpu.async_copy(b_h.at[ics[s]], bbs[s], sbs[s])

        def wait_store(i, s):
            pltpu.make_async_copy(bas[s], s_h.at[pl.ds(base + i * chunk, chunk)],
                                  sss[s]).wait()

        def step(i, s):
            s2 = (s + 2) % 4
            pltpu.make_async_copy(a_h.at[irs[s]], bas[s], sas[s]).wait()
            pltpu.make_async_copy(b_h.at[ics[s]], bbs[s], sbs[s]).wait()
            ba, bb = bas[s], bbs[s]

            def rowbody(e, carry):
                for jj in range(D // 16):
                    sl = pl.ds(jj * 16, 16)
                    ba[e, sl] = ba[e, sl] + bb[e, sl]
                return carry

            lax.fori_loop(0, chunk, rowbody, 0)
            pltpu.async_copy(ba, s_h.at[pl.ds(base + i * chunk, chunk)], sss[s])

            @pl.when(i >= 2)
            def _():
                wait_store(i - 2, s2)

            @pl.when(i + 2 < n_ch)
            def _():
                issue(i + 2, s2)

        issue(0, 0)
        issue(1, 1)

        def body(j, carry):
            i0 = 4 * j
            step(i0, 0)
            step(i0 + 1, 1)
            step(i0 + 2, 2)
            step(i0 + 3, 3)
            return carry

        lax.fori_loop(0, n_ch // 4, body, 0)
        step(n_ch - 1, (n_ch - 1) % 4)
        wait_store(n_ch - 2, (n_ch - 2) % 4)
        wait_store(n_ch - 1, (n_ch - 1) % 4)

    return k(row, col, A, B)


def _tc_edge_mlp(S, dist, wd, b1, W2, b2, tile_e):
    E, D = S.shape

    def body(s_ref, d_ref, wd_ref, b1_ref, w2_ref, b2_ref, m_ref):
        x = s_ref[...] + d_ref[...] * wd_ref[...] + b1_ref[...]
        x = _silu(x)
        y = jnp.dot(x, w2_ref[...],
                    preferred_element_type=jnp.float32) + b2_ref[...]
        m_ref[...] = _silu(y)

    return pl.pallas_call(
        body,
        grid=(E // tile_e,),
        in_specs=[
            pl.BlockSpec((tile_e, D), lambda i: (i, 0)),
            pl.BlockSpec((tile_e, 1), lambda i: (i, 0)),
            pl.BlockSpec((1, D), lambda i: (0, 0)),
            pl.BlockSpec((1, D), lambda i: (0, 0)),
            pl.BlockSpec((D, D), lambda i: (0, 0)),
            pl.BlockSpec((1, D), lambda i: (0, 0)),
        ],
        out_specs=pl.BlockSpec((tile_e, D), lambda i: (i, 0)),
        out_shape=jax.ShapeDtypeStruct((E, D), jnp.float32),
    )(S, dist, wd, b1, W2, b2)


def _sc_scatter_add(m, row, zeros, chunk):
    E, D = m.shape
    NP = zeros.shape[0]
    epw = E // _NW
    n_ch = epw // chunk
    n_pairs = n_ch // 2
    rows_per_tile = NP // _NS
    mesh = plsc.VectorSubcoreMesh(core_axis_name="c", subcore_axis_name="s")

    @functools.partial(
        pl.kernel,
        out_type=jax.ShapeDtypeStruct((_NC * NP, D), jnp.float32),
        mesh=mesh,
        scratch_types=(
            [pltpu.VMEM((chunk,), jnp.int32)] * 4
            + [pltpu.VMEM((chunk, D), jnp.float32)] * 4
            + [pltpu.VMEM_SHARED((NP, D), jnp.float32)]
            + [pltpu.SemaphoreType.DMA] * 12
        ),
    )
    def k(m_h, row_h, z_h, out_h, idx0, idx1, idx2, idx3,
          buf0, buf1, buf2, buf3, acc,
          si0, si1, si2, si3, sm0, sm1, sm2, sm3, sc0, sc1, sc2, sc3):
        cid = lax.axis_index("c")
        sid = lax.axis_index("s")
        base = (sid * _NC + cid) * epw
        idxs, bufs = [idx0, idx1, idx2, idx3], [buf0, buf1, buf2, buf3]
        sis = [si0, si1, si2, si3]
        sms = [sm0, sm1, sm2, sm3]
        scs = [sc0, sc1, sc2, sc3]
        slab = sid * rows_per_tile
        pltpu.sync_copy(z_h.at[pl.ds(slab, rows_per_tile)],
                        acc.at[pl.ds(slab, rows_per_tile)])
        plsc.subcore_barrier()

        def load(i, s):
            off = base + i * chunk
            pltpu.async_copy(row_h.at[pl.ds(off, chunk)], idxs[s], sis[s])
            pltpu.async_copy(m_h.at[pl.ds(off, chunk)], bufs[s], sms[s])

        def wait_scat(s):
            pltpu.make_async_copy(bufs[s], acc.at[idxs[s]], scs[s]).wait()

        def step(i, s):
            s2 = (s + 2) % 4
            pltpu.make_async_copy(row_h.at[pl.ds(0, chunk)], idxs[s],
                                  sis[s]).wait()
            pltpu.make_async_copy(m_h.at[pl.ds(0, chunk)], bufs[s],
                                  sms[s]).wait()

            @pl.when(i >= 2)
            def _():
                wait_scat(s2)

            @pl.when(i + 2 < n_ch)
            def _():
                load(i + 2, s2)

            pltpu.async_copy(bufs[s], acc.at[idxs[s]], scs[s], add=True)

        load(0, 0)
        load(1, 1)

        def body(j, carry):
            i0 = 4 * j
            step(i0, 0)
            step(i0 + 1, 1)
            step(i0 + 2, 2)
            step(i0 + 3, 3)
            return carry

        lax.fori_loop(0, n_ch // 4, body, 0)
        step(n_ch - 1, (n_ch - 1) % 4)
        wait_scat((n_ch - 2) % 4)
        wait_scat((n_ch - 1) % 4)
        plsc.subcore_barrier()
        pltpu.sync_copy(acc.at[pl.ds(slab, rows_per_tile)],
                        out_h.at[pl.ds(cid * NP + slab, rows_per_tile)])

    return k(m, row, zeros)


def _tc_node_mlp(h, parts, W1h, W1a, b1, W2, b2, tile_n):
    N, D = h.shape

    def body(h_ref, p0_ref, p1_ref, w1h_ref, w1a_ref, b1_ref,
             w2_ref, b2_ref, o_ref):
        agg = p0_ref[...] + p1_ref[...]
        x = (jnp.dot(h_ref[...], w1h_ref[...], preferred_element_type=jnp.float32)
             + jnp.dot(agg, w1a_ref[...], preferred_element_type=jnp.float32)
             + b1_ref[...])
        x = _silu(x)
        o_ref[...] = jnp.dot(x, w2_ref[...],
                             preferred_element_type=jnp.float32) + b2_ref[...]

    blk = pl.BlockSpec((tile_n, D), lambda i: (i, 0))
    wblk = pl.BlockSpec((D, D), lambda i: (0, 0))
    bblk = pl.BlockSpec((1, D), lambda i: (0, 0))
    return pl.pallas_call(
        body,
        grid=(N // tile_n,),
        in_specs=[blk, blk, blk, wblk, wblk, bblk, wblk, bblk],
        out_specs=blk,
        out_shape=jax.ShapeDtypeStruct((N, D), jnp.float32),
    )(h, *parts, W1h, W1a, b1, W2, b2)


def kernel(h, row, col, dist, W_e1, b_e1, W_e2, b_e2, W_n1, b_n1, W_n2, b_n2):
    N, D = h.shape
    E = row.shape[0]
    row = row.astype(jnp.int32)
    col = col.astype(jnp.int32)

    Wa = W_e1[:D]
    Wb = W_e1[D:2 * D]
    wd = W_e1[2 * D].reshape(1, D)
    b1 = b_e1.reshape(1, D)
    b2 = b_e2.reshape(1, D)
    Wn1h = W_n1[:D]
    Wn1a = W_n1[D:]
    bn1 = b_n1.reshape(1, D)
    bn2 = b_n2.reshape(1, D)

    A, B = _tc_precompute(h, Wa, Wb, tile_n=5000)
    NP = ((N + 8 * _NS - 1) // (8 * _NS)) * (8 * _NS)
    zeros = jnp.zeros((NP, D), jnp.float32)

    S = _sc_gather_add(row, col, A, B, chunk=80)
    m = _tc_edge_mlp(S, dist, wd, b1, W_e2, b2, tile_e=5000)
    P = _sc_scatter_add(m, row, zeros, chunk=80)
    parts = [P[:N], P[NP:NP + N]]
    return _tc_node_mlp(h, parts, Wn1h, Wn1a, bn1, W_n2, b2=bn2, tile_n=5000)


# speedup vs baseline: 1.0146x; 1.0146x over previous
import functools

import jax
import jax.numpy as jnp
from jax import lax
from jax.experimental import pallas as pl
from jax.experimental.pallas import tpu as pltpu
from jax.experimental.pallas import tpu_sc as plsc

_NC = 2
_NS = 16
_NW = _NC * _NS


def _silu(x):
    return x * jax.nn.sigmoid(x)


def _tc_precompute(h, Wa, Wb, tile_n):
    N, D = h.shape

    def body(h_ref, wa_ref, wb_ref, a_ref, b_ref):
        hx = h_ref[...]
        a_ref[...] = jnp.dot(hx, wa_ref[...], preferred_element_type=jnp.float32,
                             precision=jax.lax.Precision.HIGHEST)
        b_ref[...] = jnp.dot(hx, wb_ref[...], preferred_element_type=jnp.float32,
                             precision=jax.lax.Precision.HIGHEST)

    return pl.pallas_call(
        body,
        grid=(N // tile_n,),
        in_specs=[
            pl.BlockSpec((tile_n, D), lambda i: (i, 0)),
            pl.BlockSpec((D, D), lambda i: (0, 0)),
            pl.BlockSpec((D, D), lambda i: (0, 0)),
        ],
        out_specs=[
            pl.BlockSpec((tile_n, D), lambda i: (i, 0)),
            pl.BlockSpec((tile_n, D), lambda i: (i, 0)),
        ],
        out_shape=[jax.ShapeDtypeStruct((N, D), jnp.float32)] * 2,
    )(h, Wa, Wb)


def _sc_gather_add(row, col, A, B, chunk):
    E = row.shape[0]
    N, D = A.shape
    epw = E // _NW
    n_ch = epw // chunk
    mesh = plsc.VectorSubcoreMesh(core_axis_name="c", subcore_axis_name="s")

    @functools.partial(
        pl.kernel,
        out_type=jax.ShapeDtypeStruct((E, D), jnp.float32),
        mesh=mesh,
        scratch_types=(
            [pltpu.VMEM((chunk,), jnp.int32)] * 8
            + [pltpu.VMEM((chunk, D), jnp.float32)] * 8
            + [pltpu.SemaphoreType.DMA] * 12
        ),
    )
    def k(row_h, col_h, a_h, b_h, s_h,
          ir0, ir1, ir2, ir3, ic0, ic1, ic2, ic3,
          ba0, ba1, ba2, ba3, bb0, bb1, bb2, bb3,
          sa0, sa1, sa2, sa3, sb0, sb1, sb2, sb3, ss0, ss1, ss2, ss3):
        cid = lax.axis_index("c")
        sid = lax.axis_index("s")
        base = (sid * _NC + cid) * epw
        irs, ics = [ir0, ir1, ir2, ir3], [ic0, ic1, ic2, ic3]
        bas, bbs = [ba0, ba1, ba2, ba3], [bb0, bb1, bb2, bb3]
        sas, sbs = [sa0, sa1, sa2, sa3], [sb0, sb1, sb2, sb3]
        sss = [ss0, ss1, ss2, ss3]

        def issue(i, s):
            off = base + i * chunk
            pltpu.sync_copy(row_h.at[pl.ds(off, chunk)], irs[s])
            pltpu.sync_copy(col_h.at[pl.ds(off, chunk)], ics[s])
            pltpu.async_copy(a_h.at[irs[s]], bas[s], sas[s])
            pltpu.async_copy(b_h.at[ics[s]], bbs[s], sbs[s])

        def wait_store(i, s):
            pltpu.make_async_copy(bas[s], s_h.at[pl.ds(base + i * chunk, chunk)],
                                  sss[s]).wait()

        def step(i, s):
            s2 = (s + 2) % 4
            pltpu.make_async_copy(a_h.at[irs[s]], bas[s], sas[s]).wait()
            pltpu.make_async_copy(b_h.at[ics[s]], bbs[s], sbs[s]).wait()
            ba, bb = bas[s], bbs[s]

            def rowbody(e, carry):
                for jj in range(D // 16):
                    sl = pl.ds(jj * 16, 16)
                    ba[e, sl] = ba[e, sl] + bb[e, sl]
                return carry

            lax.fori_loop(0, chunk, rowbody, 0)
            pltpu.async_copy(ba, s_h.at[pl.ds(base + i * chunk, chunk)], sss[s])

            @pl.when(i >= 2)
            def _():
                wait_store(i - 2, s2)

            @pl.when(i + 2 < n_ch)
            def _():
                issue(i + 2, s2)

        issue(0, 0)
        issue(1, 1)

        def body(j, carry):
            i0 = 4 * j
            step(i0, 0)
            step(i0 + 1, 1)
            step(i0 + 2, 2)
            step(i0 + 3, 3)
            return carry

        lax.fori_loop(0, n_ch // 4, body, 0)
        step(n_ch - 1, (n_ch - 1) % 4)
        wait_store(n_ch - 2, (n_ch - 2) % 4)
        wait_store(n_ch - 1, (n_ch - 1) % 4)

    return k(row, col, A, B)


def _tc_edge_mlp(S, dist, wd, b1, W2, b2, tile_e):
    E, D = S.shape

    def body(s_ref, d_ref, wd_ref, b1_ref, w2_ref, b2_ref, m_ref):
        x = s_ref[...] + d_ref[...] * wd_ref[...] + b1_ref[...]
        x = _silu(x)
        y = jnp.dot(x, w2_ref[...],
                    preferred_element_type=jnp.float32) + b2_ref[...]
        m_ref[...] = _silu(y)

    return pl.pallas_call(
        body,
        grid=(E // tile_e,),
        in_specs=[
            pl.BlockSpec((tile_e, D), lambda i: (i, 0)),
            pl.BlockSpec((tile_e, 1), lambda i: (i, 0)),
            pl.BlockSpec((1, D), lambda i: (0, 0)),
            pl.BlockSpec((1, D), lambda i: (0, 0)),
            pl.BlockSpec((D, D), lambda i: (0, 0)),
            pl.BlockSpec((1, D), lambda i: (0, 0)),
        ],
        out_specs=pl.BlockSpec((tile_e, D), lambda i: (i, 0)),
        out_shape=jax.ShapeDtypeStruct((E, D), jnp.float32),
    )(S, dist, wd, b1, W2, b2)


def _sc_scatter_add(m, row, zeros, chunk):
    E, D = m.shape
    NP = zeros.shape[0]
    epw = E // _NW
    n_ch = epw // chunk
    n_pairs = n_ch // 2
    rows_per_tile = NP // _NS
    mesh = plsc.VectorSubcoreMesh(core_axis_name="c", subcore_axis_name="s")

    @functools.partial(
        pl.kernel,
        out_type=jax.ShapeDtypeStruct((_NC * NP, D), jnp.float32),
        mesh=mesh,
        scratch_types=(
            [pltpu.VMEM((chunk,), jnp.int32)] * 4
            + [pltpu.VMEM((chunk, D), jnp.float32)] * 4
            + [pltpu.VMEM_SHARED((NP, D), jnp.float32)]
            + [pltpu.SemaphoreType.DMA] * 12
        ),
    )
    def k(m_h, row_h, z_h, out_h, idx0, idx1, idx2, idx3,
          buf0, buf1, buf2, buf3, acc,
          si0, si1, si2, si3, sm0, sm1, sm2, sm3, sc0, sc1, sc2, sc3):
        cid = lax.axis_index("c")
        sid = lax.axis_index("s")
        base = (sid * _NC + cid) * epw
        idxs, bufs = [idx0, idx1, idx2, idx3], [buf0, buf1, buf2, buf3]
        sis = [si0, si1, si2, si3]
        sms = [sm0, sm1, sm2, sm3]
        scs = [sc0, sc1, sc2, sc3]
        slab = sid * rows_per_tile
        pltpu.sync_copy(z_h.at[pl.ds(slab, rows_per_tile)],
                        acc.at[pl.ds(slab, rows_per_tile)])
        plsc.subcore_barrier()

        def load(i, s):
            off = base + i * chunk
            pltpu.async_copy(row_h.at[pl.ds(off, chunk)], idxs[s], sis[s])
            pltpu.async_copy(m_h.at[pl.ds(off, chunk)], bufs[s], sms[s])

        def wait_scat(s):
            pltpu.make_async_copy(bufs[s], acc.at[idxs[s]], scs[s]).wait()

        def step(i, s):
            s2 = (s + 2) % 4
            pltpu.make_async_copy(row_h.at[pl.ds(0, chunk)], idxs[s],
                                  sis[s]).wait()
            pltpu.make_async_copy(m_h.at[pl.ds(0, chunk)], bufs[s],
                                  sms[s]).wait()

            @pl.when(i >= 2)
            def _():
                wait_scat(s2)

            @pl.when(i + 2 < n_ch)
            def _():
                load(i + 2, s2)

            pltpu.async_copy(bufs[s], acc.at[idxs[s]], scs[s], add=True)

        load(0, 0)
        load(1, 1)

        def body(j, carry):
            i0 = 4 * j
            step(i0, 0)
            step(i0 + 1, 1)
            step(i0 + 2, 2)
            step(i0 + 3, 3)
            return carry

        lax.fori_loop(0, n_ch // 4, body, 0)
        step(n_ch - 1, (n_ch - 1) % 4)
        wait_scat((n_ch - 2) % 4)
        wait_scat((n_ch - 1) % 4)
        plsc.subcore_barrier()
        pltpu.sync_copy(acc.at[pl.ds(slab, rows_per_tile)],
                        out_h.at[pl.ds(cid * NP + slab, rows_per_tile)])

    return k(m, row, zeros)


def _tc_node_mlp(h, parts, W1h, W1a, b1, W2, b2, tile_n):
    N, D = h.shape

    def body(h_ref, p0_ref, p1_ref, w1h_ref, w1a_ref, b1_ref,
             w2_ref, b2_ref, o_ref):
        agg = p0_ref[...] + p1_ref[...]
        x = (jnp.dot(h_ref[...], w1h_ref[...], preferred_element_type=jnp.float32)
             + jnp.dot(agg, w1a_ref[...], preferred_element_type=jnp.float32)
             + b1_ref[...])
        x = _silu(x)
        o_ref[...] = jnp.dot(x, w2_ref[...],
                             preferred_element_type=jnp.float32) + b2_ref[...]

    blk = pl.BlockSpec((tile_n, D), lambda i: (i, 0))
    wblk = pl.BlockSpec((D, D), lambda i: (0, 0))
    bblk = pl.BlockSpec((1, D), lambda i: (0, 0))
    return pl.pallas_call(
        body,
        grid=(N // tile_n,),
        in_specs=[blk, blk, blk, wblk, wblk, bblk, wblk, bblk],
        out_specs=blk,
        out_shape=jax.ShapeDtypeStruct((N, D), jnp.float32),
    )(h, *parts, W1h, W1a, b1, W2, b2)


def kernel(h, row, col, dist, W_e1, b_e1, W_e2, b_e2, W_n1, b_n1, W_n2, b_n2):
    N, D = h.shape
    E = row.shape[0]
    row = row.astype(jnp.int32)
    col = col.astype(jnp.int32)

    Wa = W_e1[:D]
    Wb = W_e1[D:2 * D]
    wd = W_e1[2 * D].reshape(1, D)
    b1 = b_e1.reshape(1, D)
    b2 = b_e2.reshape(1, D)
    Wn1h = W_n1[:D]
    Wn1a = W_n1[D:]
    bn1 = b_n1.reshape(1, D)
    bn2 = b_n2.reshape(1, D)

    A, B = _tc_precompute(h, Wa, Wb, tile_n=2000)
    NP = ((N + 8 * _NS - 1) // (8 * _NS)) * (8 * _NS)
    zeros = jnp.zeros((NP, D), jnp.float32)

    S = _sc_gather_add(row, col, A, B, chunk=80)
    m = _tc_edge_mlp(S, dist, wd, b1, W_e2, b2, tile_e=5000)
    P = _sc_scatter_add(m, row, zeros, chunk=80)
    parts = [P[:N], P[NP:NP + N]]
    return _tc_node_mlp(h, parts, Wn1h, Wn1a, bn1, W_n2, b2=bn2, tile_n=2000)
